# baseline (speedup 1.0000x reference)
import functools

import jax
import jax.numpy as jnp
from jax import lax
from jax.experimental import pallas as pl
from jax.experimental.pallas import tpu as pltpu
from jax.experimental.pallas import tpu_sc as plsc

B = 16
N_OBJ = 128
N_PRED = 32
FEAT = 64
N_ATOMS = 500000
N_NEURAL = 200000
N_BG = 50000

NC = 2
NS = 16
NW = NC * NS

TW = 4104
ONE_SLOT = 4096
ZERO_SLOT = 4097
TABLE_N = B * TW

HALF_A = 250368
ADDR_N = 251904
DUMP_LOC = 250880
INIT_CHUNK = 2048
N_INIT_CHUNKS = 123

NEUR_W = 12544
NHALF = 6272
BG_W = 3136

CH = 768
N_FULL = 651
TAIL = 32
HALF_CH = 326
N_SLOTS = 22


def _tc_table(z3, w):
    def body(z_ref, w_ref, o_ref):
        z2 = z_ref[...].reshape(B * N_OBJ, FEAT)
        s = lax.dot_general(z2, w_ref[...], (((1,), (1,)), ((), ())),
                            preferred_element_type=jnp.float32)
        o_ref[...] = jax.nn.sigmoid(s)
    return pl.pallas_call(
        body,
        out_shape=jax.ShapeDtypeStruct((B * N_OBJ, N_PRED), jnp.float32),
    )(z3, w)


def _sc_build(table, nidx, obj, prd, bg):
    i32 = jnp.int32
    f32 = jnp.float32
    mesh = plsc.VectorSubcoreMesh(core_axis_name="c", subcore_axis_name="s",
                                  num_cores=NC, num_subcores=NS)

    @functools.partial(
        pl.kernel,
        out_type=jax.ShapeDtypeStruct((B, N_ATOMS), jnp.float32),
        mesh=mesh,
        scratch_types=[
            pltpu.VMEM_SHARED((ADDR_N,), i32),
            pltpu.VMEM_SHARED((TABLE_N,), f32),
            pltpu.SemaphoreType.DMA,
            pltpu.SemaphoreType.DMA,
            pltpu.SemaphoreType.DMA,
        ],
        compiler_params=pltpu.CompilerParams(needs_layout_passes=False),
    )
    def body(table_h, nidx_h, obj_h, prd_h, bg_h, out_h, addr_sh, table_sp,
             semT, semA, semB):
        c = lax.axis_index("c")
        s = lax.axis_index("s")
        base_c = c * HALF_A
        bound_c = jnp.where(c == 0, HALF_A, N_ATOMS - HALF_A)

        @pl.when(s == 0)
        def _():
            pltpu.async_copy(table_h, table_sp, semT)

        def scatter_scope(n1, n2, n3, bgi, bgv):
            zsplat = jnp.full((16,), ZERO_SLOT, i32)
            def fill_body(i, carry):
                n1[pl.ds(i * 16, 16)] = zsplat
                return carry
            lax.fori_loop(0, INIT_CHUNK // 16, fill_body, 0)

            def init_body(i, carry):
                cid = s + NS * i
                @pl.when(cid < N_INIT_CHUNKS)
                def _():
                    pltpu.sync_copy(n1.at[pl.ds(0, INIT_CHUNK)],
                                    addr_sh.at[pl.ds(cid * INIT_CHUNK,
                                                     INIT_CHUNK)])
                return carry
            lax.fori_loop(0, 8, init_body, 0)
            plsc.subcore_barrier()

            nb = jnp.minimum(s * NEUR_W, N_NEURAL - NEUR_W)
            for h in range(2):
                base = jnp.minimum(nb + h * NHALF, N_NEURAL - NHALF)
                pltpu.sync_copy(nidx_h.at[pl.ds(base, NHALF)], n1)
                pltpu.sync_copy(obj_h.at[pl.ds(base, NHALF)], n2)
                pltpu.sync_copy(prd_h.at[pl.ds(base, NHALF)], n3)

                def comb_body(g, carry):
                    o = n2[pl.ds(g * 16, 16)]
                    p = n3[pl.ds(g * 16, 16)]
                    il = n1[pl.ds(g * 16, 16)] - base_c
                    ok = (il >= 0) & (il < bound_c)
                    n1[pl.ds(g * 16, 16)] = jnp.where(ok, il, DUMP_LOC)
                    n2[pl.ds(g * 16, 16)] = o * N_PRED + p
                    return carry
                lax.fori_loop(0, NHALF // 16, comb_body, 0)
                pltpu.sync_copy(n2, addr_sh.at[n1])

            bgb = jnp.minimum(s * BG_W, N_BG - BG_W)
            pltpu.sync_copy(bg_h.at[pl.ds(bgb, BG_W)], bgi)
            osplat = jnp.full((16,), ONE_SLOT, i32)
            def bg_body(g, carry):
                il = bgi[pl.ds(g * 16, 16)] - base_c
                ok = (il >= 0) & (il < bound_c)
                bgi[pl.ds(g * 16, 16)] = jnp.where(ok, il, DUMP_LOC)
                bgv[pl.ds(g * 16, 16)] = osplat
                return carry
            lax.fori_loop(0, BG_W // 16, bg_body, 0)
            pltpu.sync_copy(bgv, addr_sh.at[bgi])

        pl.run_scoped(scatter_scope,
                      pltpu.VMEM((NHALF,), i32),
                      pltpu.VMEM((NHALF,), i32),
                      pltpu.VMEM((NHALF,), i32),
                      pltpu.VMEM((BG_W,), i32),
                      pltpu.VMEM((BG_W,), i32))
        @pl.when(s == 0)
        def _():
            pltpu.make_async_copy(table_h, table_sp, semT).wait()
        plsc.subcore_barrier()

        def dense_scope(table_v, addr_v, outA, outB, tail_v):
            pltpu.sync_copy(table_sp, table_v)
            lane = lax.broadcasted_iota(i32, (16,), 0)

            def gather_chunk(cbase, n, buf):
                pltpu.sync_copy(addr_sh.at[pl.ds(cbase, n)],
                                addr_v.at[pl.ds(0, n)])
                def g_body(g, carry2):
                    a = addr_v[pl.ds(g * 16, 16)]
                    for b in range(B):
                        fi = a + (b * TW)
                        v = plsc.load_gather(table_v, [fi])
                        buf[b, pl.ds(g * 16, 16)] = v
                    return carry2
                lax.fori_loop(0, n // 16, g_body, 0)

            def pipe_body(j, carry):
                for k, (buf, sem) in enumerate(((outA, semA), (outB, semB))):
                    t = 2 * j + k
                    lc = s + NS * t
                    cid = c * HALF_CH + lc
                    @pl.when((t >= 2) & (lc - 2 * NS < HALF_CH)
                             & (cid - 2 * NS < N_FULL))
                    def _():
                        pltpu.make_async_copy(
                            buf, out_h.at[:, pl.ds(0, CH)], sem).wait()
                    @pl.when((lc < HALF_CH) & (cid < N_FULL))
                    def _():
                        gather_chunk(lc * CH, CH, buf)
                        @pl.when(cid == 0)
                        def _():
                            for b in range(B):
                                v = buf[b, pl.ds(0, 16)]
                                buf[b, pl.ds(0, 16)] = jnp.where(
                                    lane == 1, jnp.float32(1.0), v)
                        pltpu.async_copy(buf,
                                         out_h.at[:, pl.ds(cid * CH, CH)],
                                         sem)
                    @pl.when(cid == N_FULL)
                    def _():
                        gather_chunk((N_FULL - c * HALF_CH) * CH, TAIL,
                                     tail_v)
                        pltpu.sync_copy(tail_v,
                                        out_h.at[:, pl.ds(N_FULL * CH,
                                                          TAIL)])
                return carry
            lax.fori_loop(0, N_SLOTS // 2, pipe_body, 0)
            @pl.when((s + NS * 20 < HALF_CH)
                     & (c * HALF_CH + s + NS * 20 < N_FULL))
            def _():
                pltpu.make_async_copy(outA, out_h.at[:, pl.ds(0, CH)],
                                      semA).wait()

        pl.run_scoped(dense_scope,
                      pltpu.VMEM((TABLE_N,), f32),
                      pltpu.VMEM((CH,), i32),
                      pltpu.VMEM((B, CH), f32),
                      pltpu.VMEM((B, CH), f32),
                      pltpu.VMEM((B, TAIL), f32))

    return body(table, nidx, obj, prd, bg)


def kernel(Z, W, neural_atom_idx, atom_obj_idx, atom_pred_idx, bg_atom_idx,
           n_atoms):
    del n_atoms

    scores = _tc_table(Z, W)
    tail = jnp.tile(
        jnp.array([[1.0] + [0.0] * (TW - N_OBJ * N_PRED - 1)], jnp.float32),
        (B, 1))
    table = jnp.concatenate([scores.reshape(B, N_OBJ * N_PRED), tail],
                            axis=1).reshape(-1)
    return _sc_build(table, neural_atom_idx, atom_obj_idx, atom_pred_idx,
                     bg_atom_idx)


# speedup vs baseline: 1.9819x; 1.9819x over previous
import functools

import jax
import jax.numpy as jnp
from jax import lax
from jax.experimental import pallas as pl
from jax.experimental.pallas import tpu as pltpu
from jax.experimental.pallas import tpu_sc as plsc

B = 16
N_OBJ = 128
N_PRED = 32
FEAT = 64
N_ATOMS = 500000
N_NEURAL = 200000
N_BG = 50000

NC = 2
NS = 16
NW = NC * NS

TW = 4104
ONE_SLOT = 4096
ZERO_SLOT = 4097
TABLE_N = B * TW

ADDR_N = 501760
INIT_CHUNK = 2048
N_INIT_CHUNKS = 245

NEUR_W = 12544
NHALF = 6272
BG_W = 3136

CH = 768
N_FULL = 651
TAIL = 32
N_SLOTS = 22


def _tc_table(z3, w):
    def body(z_ref, w_ref, o_ref):
        z2 = z_ref[...].reshape(B * N_OBJ, FEAT)
        s = lax.dot_general(z2, w_ref[...], (((1,), (1,)), ((), ())),
                            preferred_element_type=jnp.float32)
        o_ref[...] = jax.nn.sigmoid(s)
    return pl.pallas_call(
        body,
        out_shape=jax.ShapeDtypeStruct((B * N_OBJ, N_PRED), jnp.float32),
    )(z3, w)


def _sc_build(table, nidx, obj, prd, bg):
    i32 = jnp.int32
    f32 = jnp.float32
    mesh = plsc.VectorSubcoreMesh(core_axis_name="c", subcore_axis_name="s",
                                  num_cores=NC, num_subcores=NS)

    @functools.partial(
        pl.kernel,
        out_type=jax.ShapeDtypeStruct((B, N_ATOMS), jnp.float32),
        mesh=mesh,
        scratch_types=[
            pltpu.VMEM_SHARED((ADDR_N,), i32),
            pltpu.VMEM_SHARED((TABLE_N,), f32),
            pltpu.SemaphoreType.DMA,
            pltpu.SemaphoreType.DMA,
            pltpu.SemaphoreType.DMA,
        ],
        compiler_params=pltpu.CompilerParams(needs_layout_passes=False),
    )
    def body(table_h, nidx_h, obj_h, prd_h, bg_h, out_h, addr_sh, table_sp,
             semT, semA, semB):
        c = lax.axis_index("c")
        s = lax.axis_index("s")
        wid = s * NC + c

        @pl.when(s == 0)
        def _():
            pltpu.async_copy(table_h, table_sp, semT)

        def scatter_scope(n1, n2, n3, bgi, bgv):
            zsplat = jnp.full((16,), ZERO_SLOT, i32)
            def fill_body(i, carry):
                n1[pl.ds(i * 16, 16)] = zsplat
                return carry
            lax.fori_loop(0, INIT_CHUNK // 16, fill_body, 0)

            def init_body(i, carry):
                cid = s + NS * i
                @pl.when(cid < N_INIT_CHUNKS)
                def _():
                    pltpu.sync_copy(n1.at[pl.ds(0, INIT_CHUNK)],
                                    addr_sh.at[pl.ds(cid * INIT_CHUNK,
                                                     INIT_CHUNK)])
                return carry
            lax.fori_loop(0, 16, init_body, 0)
            plsc.subcore_barrier()

            nb = jnp.minimum(s * NEUR_W, N_NEURAL - NEUR_W)
            for h in range(2):
                base = jnp.minimum(nb + h * NHALF, N_NEURAL - NHALF)
                pltpu.sync_copy(nidx_h.at[pl.ds(base, NHALF)], n1)
                pltpu.sync_copy(obj_h.at[pl.ds(base, NHALF)], n2)
                pltpu.sync_copy(prd_h.at[pl.ds(base, NHALF)], n3)

                def comb_body(g, carry):
                    o = n2[pl.ds(g * 16, 16)]
                    p = n3[pl.ds(g * 16, 16)]
                    n2[pl.ds(g * 16, 16)] = o * N_PRED + p
                    return carry
                lax.fori_loop(0, NHALF // 16, comb_body, 0)
                pltpu.sync_copy(n2, addr_sh.at[n1])

            bgb = jnp.minimum(s * BG_W, N_BG - BG_W)
            pltpu.sync_copy(bg_h.at[pl.ds(bgb, BG_W)], bgi)
            osplat = jnp.full((16,), ONE_SLOT, i32)
            def bg_body(g, carry):
                bgv[pl.ds(g * 16, 16)] = osplat
                return carry
            lax.fori_loop(0, BG_W // 16, bg_body, 0)
            pltpu.sync_copy(bgv, addr_sh.at[bgi])

        pl.run_scoped(scatter_scope,
                      pltpu.VMEM((NHALF,), i32),
                      pltpu.VMEM((NHALF,), i32),
                      pltpu.VMEM((NHALF,), i32),
                      pltpu.VMEM((BG_W,), i32),
                      pltpu.VMEM((BG_W,), i32))
        @pl.when(s == 0)
        def _():
            pltpu.make_async_copy(table_h, table_sp, semT).wait()
        plsc.subcore_barrier()

        def dense_scope(table_v, addr_v, outA, outB, tail_v):
            pltpu.sync_copy(table_sp, table_v)
            lane = lax.broadcasted_iota(i32, (16,), 0)

            def gather_chunk(cbase, n, buf):
                pltpu.sync_copy(addr_sh.at[pl.ds(cbase, n)],
                                addr_v.at[pl.ds(0, n)])
                def g_body(g, carry2):
                    a = addr_v[pl.ds(g * 16, 16)]
                    for b in range(B):
                        fi = a + (b * TW)
                        v = plsc.load_gather(table_v, [fi])
                        buf[b, pl.ds(g * 16, 16)] = v
                    return carry2
                lax.fori_loop(0, n // 16, g_body, 0)

            def pipe_body(j, carry):
                for k, (buf, sem) in enumerate(((outA, semA), (outB, semB))):
                    t = 2 * j + k
                    cid = wid + NW * t
                    @pl.when((t >= 2) & (cid - 2 * NW < N_FULL))
                    def _():
                        pltpu.make_async_copy(
                            buf, out_h.at[:, pl.ds(0, CH)], sem).wait()
                    @pl.when(cid < N_FULL)
                    def _():
                        gather_chunk(cid * CH, CH, buf)
                        @pl.when(cid == 0)
                        def _():
                            for b in range(B):
                                v = buf[b, pl.ds(0, 16)]
                                buf[b, pl.ds(0, 16)] = jnp.where(
                                    lane == 1, jnp.float32(1.0), v)
                        pltpu.async_copy(buf,
                                         out_h.at[:, pl.ds(cid * CH, CH)],
                                         sem)
                    @pl.when(cid == N_FULL)
                    def _():
                        gather_chunk(N_FULL * CH, TAIL, tail_v)
                        pltpu.sync_copy(tail_v,
                                        out_h.at[:, pl.ds(N_FULL * CH,
                                                          TAIL)])
                return carry
            lax.fori_loop(0, N_SLOTS // 2, pipe_body, 0)
            @pl.when(wid + NW * 20 < N_FULL)
            def _():
                pltpu.make_async_copy(outA, out_h.at[:, pl.ds(0, CH)],
                                      semA).wait()

        pl.run_scoped(dense_scope,
                      pltpu.VMEM((TABLE_N,), f32),
                      pltpu.VMEM((CH,), i32),
                      pltpu.VMEM((B, CH), f32),
                      pltpu.VMEM((B, CH), f32),
                      pltpu.VMEM((B, TAIL), f32))

    return body(table, nidx, obj, prd, bg)


def kernel(Z, W, neural_atom_idx, atom_obj_idx, atom_pred_idx, bg_atom_idx,
           n_atoms):
    del n_atoms

    scores = _tc_table(Z, W)
    tail = jnp.tile(
        jnp.array([[1.0] + [0.0] * (TW - N_OBJ * N_PRED - 1)], jnp.float32),
        (B, 1))
    table = jnp.concatenate([scores.reshape(B, N_OBJ * N_PRED), tail],
                            axis=1).reshape(-1)
    return _sc_build(table, neural_atom_idx, atom_obj_idx, atom_pred_idx,
                     bg_atom_idx)


# speedup vs baseline: 2.0927x; 1.0559x over previous
import functools

import jax
import jax.numpy as jnp
from jax import lax
from jax.experimental import pallas as pl
from jax.experimental.pallas import tpu as pltpu
from jax.experimental.pallas import tpu_sc as plsc

B = 16
N_OBJ = 128
N_PRED = 32
FEAT = 64
N_ATOMS = 500000
N_NEURAL = 200000
N_BG = 50000

NC = 2
NS = 16
NW = NC * NS

TW = 4104
ONE_SLOT = 4096
ZERO_SLOT = 4097
TABLE_N = B * TW

ADDR_N = 501760
INIT_CHUNK = 2048
N_INIT_CHUNKS = 245

NEUR_W = 12544
NHALF = 6272
BG_W = 3136

CH = 768
N_FULL = 651
TAIL = 32
N_SLOTS = 22


def _tc_table(z3, w):
    def body(z_ref, w_ref, o_ref):
        z2 = z_ref[...].reshape(B * N_OBJ, FEAT)
        s = lax.dot_general(z2, w_ref[...], (((1,), (1,)), ((), ())),
                            preferred_element_type=jnp.float32)
        o_ref[...] = jax.nn.sigmoid(s)
    return pl.pallas_call(
        body,
        out_shape=jax.ShapeDtypeStruct((B * N_OBJ, N_PRED), jnp.float32),
    )(z3, w)


def _sc_build(table, nidx, obj, prd, bg):
    i32 = jnp.int32
    f32 = jnp.float32
    mesh = plsc.VectorSubcoreMesh(core_axis_name="c", subcore_axis_name="s",
                                  num_cores=NC, num_subcores=NS)

    @functools.partial(
        pl.kernel,
        out_type=jax.ShapeDtypeStruct((B, N_ATOMS), jnp.float32),
        mesh=mesh,
        scratch_types=[
            pltpu.VMEM_SHARED((ADDR_N,), i32),
            pltpu.VMEM_SHARED((TABLE_N,), f32),
            pltpu.SemaphoreType.DMA,
            pltpu.SemaphoreType.DMA,
            pltpu.SemaphoreType.DMA,
            pltpu.SemaphoreType.DMA,
        ],
        compiler_params=pltpu.CompilerParams(needs_layout_passes=False),
    )
    def body(table_h, nidx_h, obj_h, prd_h, bg_h, out_h, addr_sh, table_sp,
             semT, semA, semB, semC):
        c = lax.axis_index("c")
        s = lax.axis_index("s")
        wid = s * NC + c

        @pl.when(s == 0)
        def _():
            pltpu.async_copy(table_h, table_sp, semT)

        def scatter_scope(n1, n2, n3, n4, n5, bgi, bgv):
            zsplat = jnp.full((16,), ZERO_SLOT, i32)
            def fill_body(i, carry):
                n1[pl.ds(i * 16, 16)] = zsplat
                return carry
            lax.fori_loop(0, INIT_CHUNK // 16, fill_body, 0)

            def init_body(i, carry):
                cid = s + NS * i
                @pl.when(cid < N_INIT_CHUNKS)
                def _():
                    pltpu.sync_copy(n1.at[pl.ds(0, INIT_CHUNK)],
                                    addr_sh.at[pl.ds(cid * INIT_CHUNK,
                                                     INIT_CHUNK)])
                return carry
            lax.fori_loop(0, 16, init_body, 0)
            plsc.subcore_barrier()

            nb = jnp.minimum(s * NEUR_W, N_NEURAL - NEUR_W)
            b0 = nb
            b1 = jnp.minimum(nb + NHALF, N_NEURAL - NHALF)

            pltpu.sync_copy(nidx_h.at[pl.ds(b0, NHALF)], n1)
            pltpu.sync_copy(obj_h.at[pl.ds(b0, NHALF)], n2)
            pltpu.sync_copy(prd_h.at[pl.ds(b0, NHALF)], n3)
            def comb_body(g, carry):
                o = n2[pl.ds(g * 16, 16)]
                p = n3[pl.ds(g * 16, 16)]
                n2[pl.ds(g * 16, 16)] = o * N_PRED + p
                return carry
            lax.fori_loop(0, NHALF // 16, comb_body, 0)
            pltpu.async_copy(n2, addr_sh.at[n1], semA)

            pltpu.sync_copy(nidx_h.at[pl.ds(b1, NHALF)], n4)
            pltpu.sync_copy(obj_h.at[pl.ds(b1, NHALF)], n5)
            pltpu.sync_copy(prd_h.at[pl.ds(b1, NHALF)], n3)
            def comb_body2(g, carry):
                o = n5[pl.ds(g * 16, 16)]
                p = n3[pl.ds(g * 16, 16)]
                n5[pl.ds(g * 16, 16)] = o * N_PRED + p
                return carry
            lax.fori_loop(0, NHALF // 16, comb_body2, 0)
            pltpu.async_copy(n5, addr_sh.at[n4], semB)

            bgb = jnp.minimum(s * BG_W, N_BG - BG_W)
            pltpu.sync_copy(bg_h.at[pl.ds(bgb, BG_W)], bgi)
            osplat = jnp.full((16,), ONE_SLOT, i32)
            def bg_body(g, carry):
                bgv[pl.ds(g * 16, 16)] = osplat
                return carry
            lax.fori_loop(0, BG_W // 16, bg_body, 0)
            pltpu.async_copy(bgv, addr_sh.at[bgi], semC)

            pltpu.make_async_copy(n2, addr_sh.at[n1], semA).wait()
            pltpu.make_async_copy(n5, addr_sh.at[n4], semB).wait()
            pltpu.make_async_copy(bgv, addr_sh.at[bgi], semC).wait()

        pl.run_scoped(scatter_scope,
                      pltpu.VMEM((NHALF,), i32),
                      pltpu.VMEM((NHALF,), i32),
                      pltpu.VMEM((NHALF,), i32),
                      pltpu.VMEM((NHALF,), i32),
                      pltpu.VMEM((NHALF,), i32),
                      pltpu.VMEM((BG_W,), i32),
                      pltpu.VMEM((BG_W,), i32))
        @pl.when(s == 0)
        def _():
            pltpu.make_async_copy(table_h, table_sp, semT).wait()
        plsc.subcore_barrier()

        def dense_scope(table_v, addr_v, outA, outB, tail_v):
            pltpu.sync_copy(table_sp, table_v)
            lane = lax.broadcasted_iota(i32, (16,), 0)

            def gather_chunk(cbase, n, buf):
                pltpu.sync_copy(addr_sh.at[pl.ds(cbase, n)],
                                addr_v.at[pl.ds(0, n)])
                def g_body(g, carry2):
                    a = addr_v[pl.ds(g * 16, 16)]
                    for b in range(B):
                        fi = a + (b * TW)
                        v = plsc.load_gather(table_v, [fi])
                        buf[b, pl.ds(g * 16, 16)] = v
                    return carry2
                lax.fori_loop(0, n // 16, g_body, 0)

            def pipe_body(j, carry):
                for k, (buf, sem) in enumerate(((outA, semA), (outB, semB))):
                    t = 2 * j + k
                    cid = wid + NW * t
                    @pl.when((t >= 2) & (cid - 2 * NW < N_FULL))
                    def _():
                        pltpu.make_async_copy(
                            buf, out_h.at[:, pl.ds(0, CH)], sem).wait()
                    @pl.when(cid < N_FULL)
                    def _():
                        gather_chunk(cid * CH, CH, buf)
                        @pl.when(cid == 0)
                        def _():
                            for b in range(B):
                                v = buf[b, pl.ds(0, 16)]
                                buf[b, pl.ds(0, 16)] = jnp.where(
                                    lane == 1, jnp.float32(1.0), v)
                        pltpu.async_copy(buf,
                                         out_h.at[:, pl.ds(cid * CH, CH)],
                                         sem)
                    @pl.when(cid == N_FULL)
                    def _():
                        gather_chunk(N_FULL * CH, TAIL, tail_v)
                        pltpu.sync_copy(tail_v,
                                        out_h.at[:, pl.ds(N_FULL * CH,
                                                          TAIL)])
                return carry
            lax.fori_loop(0, N_SLOTS // 2, pipe_body, 0)
            @pl.when(wid + NW * 20 < N_FULL)
            def _():
                pltpu.make_async_copy(outA, out_h.at[:, pl.ds(0, CH)],
                                      semA).wait()

        pl.run_scoped(dense_scope,
                      pltpu.VMEM((TABLE_N,), f32),
                      pltpu.VMEM((CH,), i32),
                      pltpu.VMEM((B, CH), f32),
                      pltpu.VMEM((B, CH), f32),
                      pltpu.VMEM((B, TAIL), f32))

    return body(table, nidx, obj, prd, bg)


def kernel(Z, W, neural_atom_idx, atom_obj_idx, atom_pred_idx, bg_atom_idx,
           n_atoms):
    del n_atoms

    scores = _tc_table(Z, W)
    tail = jnp.tile(
        jnp.array([[1.0] + [0.0] * (TW - N_OBJ * N_PRED - 1)], jnp.float32),
        (B, 1))
    table = jnp.concatenate([scores.reshape(B, N_OBJ * N_PRED), tail],
                            axis=1).reshape(-1)
    return _sc_build(table, neural_atom_idx, atom_obj_idx, atom_pred_idx,
                     bg_atom_idx)
